# Initial kernel scaffold; baseline (speedup 1.0000x reference)
import jax
import jax.numpy as jnp
from jax.experimental import pallas as pl


def kernel(x, q_w, k_w, v_w, q_scale, k_scale, v_scale, q_conv_k, q_conv_b, k_conv_k, k_conv_b, v_conv_k, v_conv_b, alpha_w, beta_w, out_w, gate_w):
    raise NotImplementedError("write your pallas kernel here")


# speedup vs baseline: 2.2214x; 2.2214x over previous
import jax
import jax.numpy as jnp
from jax.experimental import pallas as pl
from jax.experimental.pallas import tpu as pltpu

B, T, D, H, DK = 4, 2048, 1024, 16, 64
DV = 2 * DK
QKD = H * DK
VD = H * DV
KW = 4
EPS = 1e-5

C = 64
NC = T // C
TA = 256
G = 8
BH = B * H
HALO = 8
TO = 256

PREC_PROJ = jax.lax.Precision.DEFAULT
PREC_SCAN = jax.lax.Precision.HIGHEST
PREC_EXACT = jax.lax.Precision.HIGHEST
NEWTON_STEPS = 5


def _proj_kernel(xc_ref, xp_ref, qw_ref, kw_ref, vw_ref, qs_ref, ks_ref,
                 vs_ref, qck_ref, qcb_ref, kck_ref, kcb_ref, vck_ref, vcb_ref,
                 aw_ref, bw_ref,
                 q_out, k_out, v_out, g_out, b_out):
    t = pl.program_id(1)
    xc = xc_ref[0]
    xp = xp_ref[0][TA - HALO:]
    xcat = jnp.concatenate([xp, xc], axis=0)
    row = jax.lax.broadcasted_iota(jnp.int32, (TA + HALO, 1), 0)
    gtime = t * TA + row - HALO
    valid = (gtime >= 0).astype(jnp.float32)

    def qkv(w_ref, s_ref, ck_ref, cb_ref):
        y = jnp.dot(xcat, w_ref[...], preferred_element_type=jnp.float32,
                    precision=PREC_PROJ)
        mu = jnp.mean(y, axis=-1, keepdims=True)
        yc = y - mu
        y = yc * jax.lax.rsqrt(jnp.mean(yc * yc, axis=-1, keepdims=True) + EPS)
        y = y * s_ref[...] * valid
        out = cb_ref[...]
        for i in range(KW):
            out = out + y[HALO - KW + 1 + i: HALO - KW + 1 + i + TA] * ck_ref[i:i + 1]
        return out * jax.nn.sigmoid(out)

    yq = qkv(qw_ref, qs_ref, qck_ref, qcb_ref)
    yk = qkv(kw_ref, ks_ref, kck_ref, kcb_ref)
    yv = qkv(vw_ref, vs_ref, vck_ref, vcb_ref)
    for h in range(H):
        q_out[0, h] = yq[:, h * DK:(h + 1) * DK]
        k_out[0, h] = yk[:, h * DK:(h + 1) * DK]
        v_out[0, h] = yv[:, h * DV:(h + 1) * DV]

    za = jnp.dot(xc, aw_ref[...], preferred_element_type=jnp.float32,
                 precision=PREC_PROJ)
    zb = jnp.dot(xc, bw_ref[...], preferred_element_type=jnp.float32,
                 precision=PREC_PROJ)
    la = jnp.minimum(za, 0.0) - jnp.log(1.0 + jnp.exp(-jnp.abs(za)))
    be = jax.nn.sigmoid(zb)
    r = jax.lax.broadcasted_iota(jnp.int32, (TA, TA), 0)
    c_ = jax.lax.broadcasted_iota(jnp.int32, (TA, TA), 1)
    mask = ((r >= c_) & (r // C == c_ // C)).astype(jnp.float32)
    g = jnp.dot(mask, la, preferred_element_type=jnp.float32,
                precision=PREC_EXACT)
    for h in range(H):
        g_out[0, h] = g[:, h:h + 1]
        b_out[0, h] = be[:, h:h + 1]


def _scan_kernel(q_ref, k_ref, v_ref, gc_ref, bc_ref, gr_ref, br_ref,
                 o_ref, s_ref):
    c = pl.program_id(1)

    @pl.when(c == 0)
    def _init():
        s_ref[...] = jnp.zeros_like(s_ref)

    rows = jax.lax.broadcasted_iota(jnp.int32, (C, C), 0)
    cols = jax.lax.broadcasted_iota(jnp.int32, (C, C), 1)
    strict = rows > cols
    eye = (rows == cols).astype(jnp.float32)

    for gi in range(G):
        Q = q_ref[gi]
        K = k_ref[gi]
        V = v_ref[gi]
        gcol = gc_ref[gi]
        bcol = bc_ref[gi]
        grow = gr_ref[gi, pl.ds(c, 1), :]
        brow = br_ref[gi, pl.ds(c, 1), :]
        S0 = s_ref[gi]

        eg = jnp.exp(gcol)
        egC = jnp.exp(gcol[C - 1:C, :])
        dmat = jnp.where(strict, gcol - grow, -1e30)
        E = jnp.exp(dmat)

        Kb = K * bcol
        kkb = jax.lax.dot_general(K, Kb, (((1,), (1,)), ((), ())),
                                  precision=PREC_SCAN,
                                  preferred_element_type=jnp.float32)
        M = E * kkb
        X = eye - M
        for _ in range(NEWTON_STEPS):
            R = eye - X - jnp.dot(M, X, precision=PREC_SCAN,
                                  preferred_element_type=jnp.float32)
            X = X + jnp.dot(X, R, precision=PREC_SCAN,
                            preferred_element_type=jnp.float32)

        rhs = V - jnp.dot(K * eg, S0, precision=PREC_SCAN,
                          preferred_element_type=jnp.float32)
        U = jnp.dot(X, rhs, precision=PREC_SCAN,
                    preferred_element_type=jnp.float32)

        qkb = jax.lax.dot_general(Q, Kb, (((1,), (1,)), ((), ())),
                                  precision=PREC_SCAN,
                                  preferred_element_type=jnp.float32)
        W = (E + eye) * qkb
        O = jnp.dot(Q * eg, S0, precision=PREC_SCAN,
                    preferred_element_type=jnp.float32) + \
            jnp.dot(W, U, precision=PREC_SCAN,
                    preferred_element_type=jnp.float32)
        o_ref[gi] = O

        dec = jnp.exp(gcol[C - 1:C, :] - gcol)
        s_ref[gi] = egC * S0 + jax.lax.dot_general(
            Kb * dec, U, (((0,), (0,)), ((), ())),
            precision=PREC_SCAN, preferred_element_type=jnp.float32)


def _out_kernel(core_ref, x_ref, ow_ref, gw_ref, o_ref):
    core2d = jnp.concatenate([core_ref[0, h] for h in range(H)], axis=-1)
    y = jnp.dot(core2d, ow_ref[...], preferred_element_type=jnp.float32,
                precision=PREC_PROJ)
    gate = jax.nn.sigmoid(jnp.dot(x_ref[0], gw_ref[...],
                                  preferred_element_type=jnp.float32,
                                  precision=PREC_PROJ))
    o_ref[0] = y * gate


def kernel(x, q_w, k_w, v_w, q_scale, k_scale, v_scale, q_conv_k, q_conv_b,
           k_conv_k, k_conv_b, v_conv_k, v_conv_b, alpha_w, beta_w, out_w,
           gate_w):
    f32 = jnp.float32
    qs2 = q_scale.reshape(1, QKD)
    ks2 = k_scale.reshape(1, QKD)
    vs2 = v_scale.reshape(1, VD)
    qcb2 = q_conv_b.reshape(1, QKD)
    kcb2 = k_conv_b.reshape(1, QKD)
    vcb2 = v_conv_b.reshape(1, VD)

    nt = T // TA
    full = lambda shape: pl.BlockSpec(shape, lambda b, t: tuple(0 for _ in shape))
    proj_out = pl.pallas_call(
        _proj_kernel,
        grid=(B, nt),
        in_specs=[
            pl.BlockSpec((1, TA, D), lambda b, t: (b, t, 0)),
            pl.BlockSpec((1, TA, D), lambda b, t: (b, jnp.maximum(t - 1, 0), 0)),
            full((D, QKD)), full((D, QKD)), full((D, VD)),
            full((1, QKD)), full((1, QKD)), full((1, VD)),
            full((KW, QKD)), full((1, QKD)),
            full((KW, QKD)), full((1, QKD)),
            full((KW, VD)), full((1, VD)),
            full((D, H)), full((D, H)),
        ],
        out_specs=[
            pl.BlockSpec((1, H, TA, DK), lambda b, t: (b, 0, t, 0)),
            pl.BlockSpec((1, H, TA, DK), lambda b, t: (b, 0, t, 0)),
            pl.BlockSpec((1, H, TA, DV), lambda b, t: (b, 0, t, 0)),
            pl.BlockSpec((1, H, TA, 1), lambda b, t: (b, 0, t, 0)),
            pl.BlockSpec((1, H, TA, 1), lambda b, t: (b, 0, t, 0)),
        ],
        out_shape=[
            jax.ShapeDtypeStruct((B, H, T, DK), f32),
            jax.ShapeDtypeStruct((B, H, T, DK), f32),
            jax.ShapeDtypeStruct((B, H, T, DV), f32),
            jax.ShapeDtypeStruct((B, H, T, 1), f32),
            jax.ShapeDtypeStruct((B, H, T, 1), f32),
        ],
        compiler_params=pltpu.CompilerParams(
            dimension_semantics=("parallel", "arbitrary")),
        name="gdn_proj",
    )(x, x, q_w, k_w, v_w, qs2, ks2, vs2, q_conv_k, qcb2, k_conv_k, kcb2,
      v_conv_k, vcb2, alpha_w, beta_w)
    q4, k4, v4, g4, b4 = proj_out

    q3 = q4.reshape(BH, T, DK)
    k3 = k4.reshape(BH, T, DK)
    v3 = v4.reshape(BH, T, DV)
    gcol = g4.reshape(BH, T, 1)
    bcol = b4.reshape(BH, T, 1)
    grow = g4.reshape(BH, NC, C)
    brow = b4.reshape(BH, NC, C)

    o3 = pl.pallas_call(
        _scan_kernel,
        grid=(BH // G, NC),
        in_specs=[
            pl.BlockSpec((G, C, DK), lambda i, c: (i, c, 0)),
            pl.BlockSpec((G, C, DK), lambda i, c: (i, c, 0)),
            pl.BlockSpec((G, C, DV), lambda i, c: (i, c, 0)),
            pl.BlockSpec((G, C, 1), lambda i, c: (i, c, 0)),
            pl.BlockSpec((G, C, 1), lambda i, c: (i, c, 0)),
            pl.BlockSpec((G, NC, C), lambda i, c: (i, 0, 0)),
            pl.BlockSpec((G, NC, C), lambda i, c: (i, 0, 0)),
        ],
        out_specs=pl.BlockSpec((G, C, DV), lambda i, c: (i, c, 0)),
        out_shape=jax.ShapeDtypeStruct((BH, T, DV), f32),
        scratch_shapes=[pltpu.VMEM((G, DK, DV), f32)],
        compiler_params=pltpu.CompilerParams(
            dimension_semantics=("parallel", "arbitrary")),
        name="gdn_scan",
    )(q3, k3, v3, gcol, bcol, grow, brow)

    core = o3.reshape(B, H, T, DV)
    nt2 = T // TO
    out = pl.pallas_call(
        _out_kernel,
        grid=(B, nt2),
        in_specs=[
            pl.BlockSpec((1, H, TO, DV), lambda b, t: (b, 0, t, 0)),
            pl.BlockSpec((1, TO, D), lambda b, t: (b, t, 0)),
            pl.BlockSpec((VD, D), lambda b, t: (0, 0)),
            pl.BlockSpec((D, D), lambda b, t: (0, 0)),
        ],
        out_specs=pl.BlockSpec((1, TO, D), lambda b, t: (b, t, 0)),
        out_shape=jax.ShapeDtypeStruct((B, T, D), f32),
        compiler_params=pltpu.CompilerParams(
            dimension_semantics=("parallel", "arbitrary")),
        name="gdn_out",
    )(core, x, out_w, gate_w)
    return out


# speedup vs baseline: 3.1783x; 1.4308x over previous
import jax
import jax.numpy as jnp
from jax.experimental import pallas as pl
from jax.experimental.pallas import tpu as pltpu

B, T, D, H, DK = 4, 2048, 1024, 16, 64
DV = 2 * DK
QKD = H * DK
VD = H * DV
KW = 4
EPS = 1e-5

C = 64
NC = T // C
TA = 256
G = 8
BH = B * H
HALO = 8
TO = 256

PREC_PROJ = jax.lax.Precision.DEFAULT
PREC_SCAN = jax.lax.Precision.DEFAULT
PREC_EXACT = jax.lax.Precision.HIGHEST
NEWTON_STEPS = 5


def _proj_kernel(xc_ref, xp_ref, qw_ref, kw_ref, vw_ref, qs_ref, ks_ref,
                 vs_ref, qck_ref, qcb_ref, kck_ref, kcb_ref, vck_ref, vcb_ref,
                 aw_ref, bw_ref,
                 q_out, k_out, v_out, g_out, b_out):
    t = pl.program_id(1)
    xc = xc_ref[0]
    xp = xp_ref[0][TA - HALO:]
    xcat = jnp.concatenate([xp, xc], axis=0)
    row = jax.lax.broadcasted_iota(jnp.int32, (TA + HALO, 1), 0)
    gtime = t * TA + row - HALO
    valid = (gtime >= 0).astype(jnp.float32)

    def qkv(w_ref, s_ref, ck_ref, cb_ref):
        y = jnp.dot(xcat, w_ref[...], preferred_element_type=jnp.float32,
                    precision=PREC_PROJ)
        mu = jnp.mean(y, axis=-1, keepdims=True)
        yc = y - mu
        y = yc * jax.lax.rsqrt(jnp.mean(yc * yc, axis=-1, keepdims=True) + EPS)
        y = y * s_ref[...] * valid
        out = cb_ref[...]
        for i in range(KW):
            out = out + y[HALO - KW + 1 + i: HALO - KW + 1 + i + TA] * ck_ref[i:i + 1]
        return out * jax.nn.sigmoid(out)

    yq = qkv(qw_ref, qs_ref, qck_ref, qcb_ref)
    yk = qkv(kw_ref, ks_ref, kck_ref, kcb_ref)
    yv = qkv(vw_ref, vs_ref, vck_ref, vcb_ref)
    for h in range(H):
        q_out[0, h] = yq[:, h * DK:(h + 1) * DK]
        k_out[0, h] = yk[:, h * DK:(h + 1) * DK]
        v_out[0, h] = yv[:, h * DV:(h + 1) * DV]

    za = jnp.dot(xc, aw_ref[...], preferred_element_type=jnp.float32,
                 precision=PREC_PROJ)
    zb = jnp.dot(xc, bw_ref[...], preferred_element_type=jnp.float32,
                 precision=PREC_PROJ)
    la = jnp.minimum(za, 0.0) - jnp.log(1.0 + jnp.exp(-jnp.abs(za)))
    be = jax.nn.sigmoid(zb)
    r = jax.lax.broadcasted_iota(jnp.int32, (TA, TA), 0)
    c_ = jax.lax.broadcasted_iota(jnp.int32, (TA, TA), 1)
    mask = ((r >= c_) & (r // C == c_ // C)).astype(jnp.float32)
    g = jnp.dot(mask, la, preferred_element_type=jnp.float32,
                precision=PREC_EXACT)
    for h in range(H):
        g_out[0, h] = g[:, h:h + 1]
        b_out[0, h] = be[:, h:h + 1]


def _scan_kernel(q_ref, k_ref, v_ref, gc_ref, bc_ref, gr_ref, br_ref,
                 o_ref, s_ref):
    c = pl.program_id(1)

    @pl.when(c == 0)
    def _init():
        s_ref[...] = jnp.zeros_like(s_ref)

    rows = jax.lax.broadcasted_iota(jnp.int32, (C, C), 0)
    cols = jax.lax.broadcasted_iota(jnp.int32, (C, C), 1)
    strict = rows > cols
    eye = (rows == cols).astype(jnp.float32)

    for gi in range(G):
        Q = q_ref[gi]
        K = k_ref[gi]
        V = v_ref[gi]
        gcol = gc_ref[gi]
        bcol = bc_ref[gi]
        grow = gr_ref[gi, pl.ds(c, 1), :]
        brow = br_ref[gi, pl.ds(c, 1), :]
        S0 = s_ref[gi]

        eg = jnp.exp(gcol)
        egC = jnp.exp(gcol[C - 1:C, :])
        dmat = jnp.where(strict, gcol - grow, -1e30)
        E = jnp.exp(dmat)

        Kb = K * bcol
        kkb = jax.lax.dot_general(K, Kb, (((1,), (1,)), ((), ())),
                                  precision=PREC_SCAN,
                                  preferred_element_type=jnp.float32)
        M = E * kkb
        X = eye - M
        for _ in range(NEWTON_STEPS):
            R = eye - X - jnp.dot(M, X, precision=PREC_SCAN,
                                  preferred_element_type=jnp.float32)
            X = X + jnp.dot(X, R, precision=PREC_SCAN,
                            preferred_element_type=jnp.float32)

        rhs = V - jnp.dot(K * eg, S0, precision=PREC_SCAN,
                          preferred_element_type=jnp.float32)
        U = jnp.dot(X, rhs, precision=PREC_SCAN,
                    preferred_element_type=jnp.float32)

        qkb = jax.lax.dot_general(Q, Kb, (((1,), (1,)), ((), ())),
                                  precision=PREC_SCAN,
                                  preferred_element_type=jnp.float32)
        W = (E + eye) * qkb
        O = jnp.dot(Q * eg, S0, precision=PREC_SCAN,
                    preferred_element_type=jnp.float32) + \
            jnp.dot(W, U, precision=PREC_SCAN,
                    preferred_element_type=jnp.float32)
        o_ref[gi] = O

        dec = jnp.exp(gcol[C - 1:C, :] - gcol)
        s_ref[gi] = egC * S0 + jax.lax.dot_general(
            Kb * dec, U, (((0,), (0,)), ((), ())),
            precision=PREC_SCAN, preferred_element_type=jnp.float32)


def _out_kernel(core_ref, x_ref, ow_ref, gw_ref, o_ref):
    core2d = jnp.concatenate([core_ref[0, h] for h in range(H)], axis=-1)
    y = jnp.dot(core2d, ow_ref[...], preferred_element_type=jnp.float32,
                precision=PREC_PROJ)
    gate = jax.nn.sigmoid(jnp.dot(x_ref[0], gw_ref[...],
                                  preferred_element_type=jnp.float32,
                                  precision=PREC_PROJ))
    o_ref[0] = y * gate


def kernel(x, q_w, k_w, v_w, q_scale, k_scale, v_scale, q_conv_k, q_conv_b,
           k_conv_k, k_conv_b, v_conv_k, v_conv_b, alpha_w, beta_w, out_w,
           gate_w):
    f32 = jnp.float32
    qs2 = q_scale.reshape(1, QKD)
    ks2 = k_scale.reshape(1, QKD)
    vs2 = v_scale.reshape(1, VD)
    qcb2 = q_conv_b.reshape(1, QKD)
    kcb2 = k_conv_b.reshape(1, QKD)
    vcb2 = v_conv_b.reshape(1, VD)

    nt = T // TA
    full = lambda shape: pl.BlockSpec(shape, lambda b, t: tuple(0 for _ in shape))
    proj_out = pl.pallas_call(
        _proj_kernel,
        grid=(B, nt),
        in_specs=[
            pl.BlockSpec((1, TA, D), lambda b, t: (b, t, 0)),
            pl.BlockSpec((1, TA, D), lambda b, t: (b, jnp.maximum(t - 1, 0), 0)),
            full((D, QKD)), full((D, QKD)), full((D, VD)),
            full((1, QKD)), full((1, QKD)), full((1, VD)),
            full((KW, QKD)), full((1, QKD)),
            full((KW, QKD)), full((1, QKD)),
            full((KW, VD)), full((1, VD)),
            full((D, H)), full((D, H)),
        ],
        out_specs=[
            pl.BlockSpec((1, H, TA, DK), lambda b, t: (b, 0, t, 0)),
            pl.BlockSpec((1, H, TA, DK), lambda b, t: (b, 0, t, 0)),
            pl.BlockSpec((1, H, TA, DV), lambda b, t: (b, 0, t, 0)),
            pl.BlockSpec((1, H, TA, 1), lambda b, t: (b, 0, t, 0)),
            pl.BlockSpec((1, H, TA, 1), lambda b, t: (b, 0, t, 0)),
        ],
        out_shape=[
            jax.ShapeDtypeStruct((B, H, T, DK), f32),
            jax.ShapeDtypeStruct((B, H, T, DK), f32),
            jax.ShapeDtypeStruct((B, H, T, DV), f32),
            jax.ShapeDtypeStruct((B, H, T, 1), f32),
            jax.ShapeDtypeStruct((B, H, T, 1), f32),
        ],
        compiler_params=pltpu.CompilerParams(
            dimension_semantics=("parallel", "arbitrary")),
        name="gdn_proj",
    )(x, x, q_w, k_w, v_w, qs2, ks2, vs2, q_conv_k, qcb2, k_conv_k, kcb2,
      v_conv_k, vcb2, alpha_w, beta_w)
    q4, k4, v4, g4, b4 = proj_out

    q3 = q4.reshape(BH, T, DK)
    k3 = k4.reshape(BH, T, DK)
    v3 = v4.reshape(BH, T, DV)
    gcol = g4.reshape(BH, T, 1)
    bcol = b4.reshape(BH, T, 1)
    grow = g4.reshape(BH, NC, C)
    brow = b4.reshape(BH, NC, C)

    o3 = pl.pallas_call(
        _scan_kernel,
        grid=(BH // G, NC),
        in_specs=[
            pl.BlockSpec((G, C, DK), lambda i, c: (i, c, 0)),
            pl.BlockSpec((G, C, DK), lambda i, c: (i, c, 0)),
            pl.BlockSpec((G, C, DV), lambda i, c: (i, c, 0)),
            pl.BlockSpec((G, C, 1), lambda i, c: (i, c, 0)),
            pl.BlockSpec((G, C, 1), lambda i, c: (i, c, 0)),
            pl.BlockSpec((G, NC, C), lambda i, c: (i, 0, 0)),
            pl.BlockSpec((G, NC, C), lambda i, c: (i, 0, 0)),
        ],
        out_specs=pl.BlockSpec((G, C, DV), lambda i, c: (i, c, 0)),
        out_shape=jax.ShapeDtypeStruct((BH, T, DV), f32),
        scratch_shapes=[pltpu.VMEM((G, DK, DV), f32)],
        compiler_params=pltpu.CompilerParams(
            dimension_semantics=("parallel", "arbitrary")),
        name="gdn_scan",
    )(q3, k3, v3, gcol, bcol, grow, brow)

    core = o3.reshape(B, H, T, DV)
    nt2 = T // TO
    out = pl.pallas_call(
        _out_kernel,
        grid=(B, nt2),
        in_specs=[
            pl.BlockSpec((1, H, TO, DV), lambda b, t: (b, 0, t, 0)),
            pl.BlockSpec((1, TO, D), lambda b, t: (b, t, 0)),
            pl.BlockSpec((VD, D), lambda b, t: (0, 0)),
            pl.BlockSpec((D, D), lambda b, t: (0, 0)),
        ],
        out_specs=pl.BlockSpec((1, TO, D), lambda b, t: (b, t, 0)),
        out_shape=jax.ShapeDtypeStruct((B, T, D), f32),
        compiler_params=pltpu.CompilerParams(
            dimension_semantics=("parallel", "arbitrary")),
        name="gdn_out",
    )(core, x, out_w, gate_w)
    return out


# speedup vs baseline: 17.7450x; 5.5831x over previous
import jax
import jax.numpy as jnp
from jax.experimental import pallas as pl
from jax.experimental.pallas import tpu as pltpu

B, T, D, H, DK = 4, 2048, 1024, 16, 64
DV = 2 * DK
QKD = H * DK
VD = H * DV
KW = 4
EPS = 1e-5

C = 64
NC = T // C
TA = 256
G = 32
BH = B * H
HALO = 8
TO = 256

PREC_PROJ = jax.lax.Precision.DEFAULT
PREC_SCAN = jax.lax.Precision.DEFAULT
PREC_EXACT = jax.lax.Precision.HIGHEST
NEWTON_STEPS = 5


def _proj_kernel(xc_ref, xp_ref, qw_ref, kw_ref, vw_ref, qs_ref, ks_ref,
                 vs_ref, qck_ref, qcb_ref, kck_ref, kcb_ref, vck_ref, vcb_ref,
                 aw_ref, bw_ref,
                 q_out, k_out, v_out, g_out, b_out):
    t = pl.program_id(2)
    xc = xc_ref[0]
    xp = xp_ref[0][TA - HALO:]
    xcat = jnp.concatenate([xp, xc], axis=0).astype(jnp.bfloat16)
    row = jax.lax.broadcasted_iota(jnp.int32, (TA + HALO, 1), 0)
    gtime = t * TA + row - HALO
    valid = (gtime >= 0).astype(jnp.float32)

    def qkv(w_ref, s_ref, ck_ref, cb_ref):
        y = jnp.dot(xcat, w_ref[...], preferred_element_type=jnp.float32)
        mu = jnp.mean(y, axis=-1, keepdims=True)
        yc = y - mu
        f = jax.lax.rsqrt(jnp.mean(yc * yc, axis=-1, keepdims=True) + EPS) * valid
        y = yc * f * s_ref[...]
        out = cb_ref[...]
        for i in range(KW):
            out = out + y[HALO - KW + 1 + i: HALO - KW + 1 + i + TA] * ck_ref[i:i + 1]
        return out * jax.nn.sigmoid(out)

    yq = qkv(qw_ref, qs_ref, qck_ref, qcb_ref)
    yk = qkv(kw_ref, ks_ref, kck_ref, kcb_ref)
    yv = qkv(vw_ref, vs_ref, vck_ref, vcb_ref)
    for h in range(H):
        q_out[0, h] = yq[:, h * DK:(h + 1) * DK]
        k_out[0, h] = yk[:, h * DK:(h + 1) * DK]
        v_out[0, h] = yv[:, h * DV:(h + 1) * DV]

    xcb = xcat[HALO:]
    za = jnp.dot(xcb, aw_ref[...], preferred_element_type=jnp.float32)
    zb = jnp.dot(xcb, bw_ref[...], preferred_element_type=jnp.float32)
    la = jnp.minimum(za, 0.0) - jnp.log(1.0 + jnp.exp(-jnp.abs(za)))
    be = jax.nn.sigmoid(zb)
    r = jax.lax.broadcasted_iota(jnp.int32, (TA, TA), 0)
    c_ = jax.lax.broadcasted_iota(jnp.int32, (TA, TA), 1)
    mask = ((r >= c_) & (r // C == c_ // C)).astype(jnp.bfloat16)
    la_hi = la.astype(jnp.bfloat16)
    la_lo = (la - la_hi.astype(jnp.float32)).astype(jnp.bfloat16)
    g = (jnp.dot(mask, la_hi, preferred_element_type=jnp.float32) +
         jnp.dot(mask, la_lo, preferred_element_type=jnp.float32))
    for h in range(H):
        g_out[0, h] = g[:, h:h + 1]
        b_out[0, h] = be[:, h:h + 1]


def _scan_kernel(q_ref, k_ref, v_ref, gc_ref, bc_ref, gr_ref, br_ref,
                 o_ref, s_ref):
    c = pl.program_id(1)

    @pl.when(c == 0)
    def _init():
        s_ref[...] = jnp.zeros_like(s_ref)

    rows = jax.lax.broadcasted_iota(jnp.int32, (C, C), 0)
    cols = jax.lax.broadcasted_iota(jnp.int32, (C, C), 1)
    strict = rows > cols
    eye = (rows == cols).astype(jnp.float32)

    def dot_(a, b):
        return jnp.dot(a, b, precision=PREC_SCAN,
                       preferred_element_type=jnp.float32)

    def dot_nt(a, b):
        return jax.lax.dot_general(a, b, (((1,), (1,)), ((), ())),
                                   precision=PREC_SCAN,
                                   preferred_element_type=jnp.float32)

    def dot_tn(a, b):
        return jax.lax.dot_general(a, b, (((0,), (0,)), ((), ())),
                                   precision=PREC_SCAN,
                                   preferred_element_type=jnp.float32)

    rng = range(G)
    Ks = [k_ref[gi] for gi in rng]
    Qs = [q_ref[gi] for gi in rng]
    gcols = [gc_ref[gi] for gi in rng]
    bcols = [bc_ref[gi] for gi in rng]
    egs = [jnp.exp(g) for g in gcols]
    egCs = [jnp.exp(g[C - 1:C, :]) for g in gcols]
    decs = [jnp.exp(g[C - 1:C, :] - g) for g in gcols]
    Es = [jnp.exp(jnp.where(strict, gcols[gi] - gr_ref[gi, pl.ds(c, 1), :],
                            -1e30)) for gi in rng]
    Kbs = [Ks[gi] * bcols[gi] for gi in rng]

    cat_kq = [jnp.concatenate([Ks[gi], Qs[gi]], axis=0) for gi in rng]
    gram = [dot_nt(cat_kq[gi], Kbs[gi]) for gi in rng]
    Ms = [Es[gi] * gram[gi][:C] for gi in rng]
    Ws = [(Es[gi] + eye) * gram[gi][C:] for gi in rng]

    cat_g = [cat_kq[gi] * jnp.concatenate([egs[gi], egs[gi]], axis=0)
             for gi in rng]
    readout = [dot_(cat_g[gi], s_ref[gi]) for gi in rng]
    rhss = [v_ref[gi] - readout[gi][:C] for gi in rng]
    QS0s = [readout[gi][C:] for gi in rng]

    Xs = [eye - M for M in Ms]
    for _ in range(NEWTON_STEPS):
        MXs = [dot_(Ms[gi], Xs[gi]) for gi in rng]
        Rs = [eye - Xs[gi] - MXs[gi] for gi in rng]
        XRs = [dot_(Xs[gi], Rs[gi]) for gi in rng]
        Xs = [Xs[gi] + XRs[gi] for gi in rng]

    Us = [dot_(Xs[gi], rhss[gi]) for gi in rng]
    WUs = [dot_(Ws[gi], Us[gi]) for gi in rng]
    Snews = [egCs[gi] * s_ref[gi] + dot_tn(Kbs[gi] * decs[gi], Us[gi])
             for gi in rng]
    for gi in rng:
        o_ref[gi] = QS0s[gi] + WUs[gi]
    for gi in rng:
        s_ref[gi] = Snews[gi]


def _out_kernel(core_ref, x_ref, ow_ref, gw_ref, o_ref):
    core2d = jnp.concatenate(
        [core_ref[0, h] for h in range(H)], axis=-1).astype(jnp.bfloat16)
    y = jnp.dot(core2d, ow_ref[...], preferred_element_type=jnp.float32)
    gate = jax.nn.sigmoid(
        jnp.dot(x_ref[0].astype(jnp.bfloat16), gw_ref[...],
                preferred_element_type=jnp.float32))
    o_ref[0] = y * gate


def kernel(x, q_w, k_w, v_w, q_scale, k_scale, v_scale, q_conv_k, q_conv_b,
           k_conv_k, k_conv_b, v_conv_k, v_conv_b, alpha_w, beta_w, out_w,
           gate_w):
    f32 = jnp.float32
    bf16 = jnp.bfloat16
    qwb = q_w.astype(bf16)
    kwb = k_w.astype(bf16)
    vwb = v_w.astype(bf16)
    awb = alpha_w.astype(bf16)
    bwb = beta_w.astype(bf16)
    owb = out_w.astype(bf16)
    gwb = gate_w.astype(bf16)
    qs2 = q_scale.reshape(1, QKD)
    ks2 = k_scale.reshape(1, QKD)
    vs2 = v_scale.reshape(1, VD)
    qcb2 = q_conv_b.reshape(1, QKD)
    kcb2 = k_conv_b.reshape(1, QKD)
    vcb2 = v_conv_b.reshape(1, VD)

    nt = T // TA
    B2 = B // 2
    full = lambda shape: pl.BlockSpec(shape, lambda p, b, t: tuple(0 for _ in shape))
    proj_out = pl.pallas_call(
        _proj_kernel,
        grid=(2, B2, nt),
        in_specs=[
            pl.BlockSpec((1, TA, D), lambda p, b, t: (p * (B // 2) + b, t, 0)),
            pl.BlockSpec((1, TA, D), lambda p, b, t: (p * (B // 2) + b, jnp.maximum(t - 1, 0), 0)),
            full((D, QKD)), full((D, QKD)), full((D, VD)),
            full((1, QKD)), full((1, QKD)), full((1, VD)),
            full((KW, QKD)), full((1, QKD)),
            full((KW, QKD)), full((1, QKD)),
            full((KW, VD)), full((1, VD)),
            full((D, H)), full((D, H)),
        ],
        out_specs=[
            pl.BlockSpec((1, H, TA, DK), lambda p, b, t: (p * (B // 2) + b, 0, t, 0)),
            pl.BlockSpec((1, H, TA, DK), lambda p, b, t: (p * (B // 2) + b, 0, t, 0)),
            pl.BlockSpec((1, H, TA, DV), lambda p, b, t: (p * (B // 2) + b, 0, t, 0)),
            pl.BlockSpec((1, H, TA, 1), lambda p, b, t: (p * (B // 2) + b, 0, t, 0)),
            pl.BlockSpec((1, H, TA, 1), lambda p, b, t: (p * (B // 2) + b, 0, t, 0)),
        ],
        out_shape=[
            jax.ShapeDtypeStruct((B, H, T, DK), f32),
            jax.ShapeDtypeStruct((B, H, T, DK), f32),
            jax.ShapeDtypeStruct((B, H, T, DV), f32),
            jax.ShapeDtypeStruct((B, H, T, 1), f32),
            jax.ShapeDtypeStruct((B, H, T, 1), f32),
        ],
        compiler_params=pltpu.CompilerParams(
            dimension_semantics=("parallel", "arbitrary", "arbitrary")),
        name="gdn_proj",
    )(x, x, qwb, kwb, vwb, qs2, ks2, vs2, q_conv_k, qcb2, k_conv_k, kcb2,
      v_conv_k, vcb2, awb, bwb)
    q4, k4, v4, g4, b4 = proj_out

    q3 = q4.reshape(BH, T, DK)
    k3 = k4.reshape(BH, T, DK)
    v3 = v4.reshape(BH, T, DV)
    gcol = g4.reshape(BH, T, 1)
    bcol = b4.reshape(BH, T, 1)
    grow = g4.reshape(BH, NC, C)
    brow = b4.reshape(BH, NC, C)

    o3 = pl.pallas_call(
        _scan_kernel,
        grid=(BH // G, NC),
        in_specs=[
            pl.BlockSpec((G, C, DK), lambda i, c: (i, c, 0)),
            pl.BlockSpec((G, C, DK), lambda i, c: (i, c, 0)),
            pl.BlockSpec((G, C, DV), lambda i, c: (i, c, 0)),
            pl.BlockSpec((G, C, 1), lambda i, c: (i, c, 0)),
            pl.BlockSpec((G, C, 1), lambda i, c: (i, c, 0)),
            pl.BlockSpec((G, NC, C), lambda i, c: (i, 0, 0)),
            pl.BlockSpec((G, NC, C), lambda i, c: (i, 0, 0)),
        ],
        out_specs=pl.BlockSpec((G, C, DV), lambda i, c: (i, c, 0)),
        out_shape=jax.ShapeDtypeStruct((BH, T, DV), f32),
        scratch_shapes=[pltpu.VMEM((G, DK, DV), f32)],
        compiler_params=pltpu.CompilerParams(
            dimension_semantics=("parallel", "arbitrary")),
        name="gdn_scan",
    )(q3, k3, v3, gcol, bcol, grow, brow)

    core = o3.reshape(B, H, T, DV)
    nt2 = T // TO
    out = pl.pallas_call(
        _out_kernel,
        grid=(2, B2, nt2),
        in_specs=[
            pl.BlockSpec((1, H, TO, DV), lambda p, b, t: (p * (B // 2) + b, 0, t, 0)),
            pl.BlockSpec((1, TO, D), lambda p, b, t: (p * (B // 2) + b, t, 0)),
            pl.BlockSpec((VD, D), lambda p, b, t: (0, 0)),
            pl.BlockSpec((D, D), lambda p, b, t: (0, 0)),
        ],
        out_specs=pl.BlockSpec((1, TO, D), lambda p, b, t: (p * (B // 2) + b, t, 0)),
        out_shape=jax.ShapeDtypeStruct((B, T, D), f32),
        compiler_params=pltpu.CompilerParams(
            dimension_semantics=("parallel", "arbitrary", "arbitrary")),
        name="gdn_out",
    )(core, x, owb, gwb)
    return out
